# baseline (speedup 1.0000x reference)
import jax
import jax.numpy as jnp
from jax.experimental import pallas as pl
from jax.experimental.pallas import tpu as pltpu

N = 10000
T = 2000
M = 128
D = 256


def _spmm(adj_index, adj_value, x, n):
    row, col = adj_index[0], adj_index[1]
    return jax.ops.segment_sum(adj_value[:, None] * x[col], row, num_segments=n)


def _gat(x_src, x_dst, src, dst, Ws, Wd, a_s, a_d, b, n_dst):
    hs = x_src @ Ws
    hd = x_dst @ Wd
    e = jax.nn.leaky_relu((hs @ a_s)[src] + (hd @ a_d)[dst], 0.01)
    emax = jax.ops.segment_max(e, dst, num_segments=n_dst)
    emax = jnp.where(jnp.isfinite(emax), emax, 0.0)
    a = jnp.exp(e - emax[dst])
    denom = jax.ops.segment_sum(a, dst, num_segments=n_dst)
    alpha = a / (denom[dst] + 1e-16)
    out = jax.ops.segment_sum(alpha[:, None] * hs[src], dst, num_segments=n_dst)
    return out + b


def _mlp_kernel(x_ref, p1_ref, pb1_ref, p2_ref, pb2_ref, o_ref):
    p = jnp.maximum(x_ref[...] @ p1_ref[...] + pb1_ref[...], 0.0)
    o_ref[...] = p @ p2_ref[...] + pb2_ref[...]


def kernel(node_attr, adj_index, adj_value, tt_node_batch, tt_graph_batch,
           W1, b1, W2, b2,
           g1Ws, g1Wd, g1as, g1ad, g1b,
           g2Ws, g2Wd, g2as, g2ad, g2b,
           P1, pb1, P2, pb2):
    n = node_attr.shape[0]
    t = tt_graph_batch.shape[0]
    h = jax.nn.relu(_spmm(adj_index, adj_value, node_attr, n) @ W1 + b1)
    h1 = jax.nn.relu(_spmm(adj_index, adj_value, h, n) @ W2 + b2)
    tt_attr = jax.nn.relu(jax.ops.segment_sum(h1, tt_node_batch, num_segments=t))
    src = jnp.arange(n)
    tt_embd = jax.nn.relu(jax.nn.elu(_gat(h1, tt_attr, src, tt_node_batch, g1Ws, g1Wd, g1as, g1ad, g1b, t)))
    mol_attr = jax.nn.relu(jax.ops.segment_sum(tt_embd, tt_graph_batch, num_segments=M))
    src2 = jnp.arange(t)
    mol_embd = jax.nn.relu(jax.nn.elu(_gat(tt_embd, mol_attr, src2, tt_graph_batch, g2Ws, g2Wd, g2as, g2ad, g2b, M)))
    y = pl.pallas_call(
        _mlp_kernel,
        out_shape=jax.ShapeDtypeStruct((M, 1), jnp.float32),
    )(mol_embd, P1, pb1.reshape(1, 128), P2, pb2.reshape(1, 1))
    return y


# speedup vs baseline: 2.7334x; 2.7334x over previous
import functools

import jax
import jax.numpy as jnp
from jax import lax
from jax.experimental import pallas as pl
from jax.experimental.pallas import tpu as pltpu
from jax.experimental.pallas import tpu_sc as plsc

N = 10000
E = 160000
D = 256
T = 2000
M = 128

NP_ = 10240
TP_ = 2048
BLK = 256

NS = 16
EPT = E // NS
CCH = 80
NCHUNK = EPT // CCH
RPT = NP_ // NS
ZR = 64


def _spmm_body(xlo_hbm, xhi_hbm, row_hbm, col_hbm, val_hbm, out_lo, out_hi,
               acc, zbuf, colv, rowv, valv, rows_v, sem):
    c = lax.axis_index("c")
    s = lax.axis_index("s")

    def zrow(i, carry):
        for j in range(8):
            zbuf[i, pl.ds(j * 16, 16)] = jnp.zeros((16,), jnp.float32)
        return carry

    lax.fori_loop(0, ZR, zrow, 0)
    for k in range(RPT // ZR):
        pltpu.sync_copy(zbuf, acc.at[pl.ds(s * RPT + k * ZR, ZR)])
    plsc.subcore_barrier()

    def half(x_hbm, out_hbm):
        def chunk(i, carry):
            base = s * EPT + i * CCH
            pltpu.sync_copy(col_hbm.at[pl.ds(base, CCH)], colv)
            pltpu.sync_copy(row_hbm.at[pl.ds(base, CCH)], rowv)
            pltpu.sync_copy(val_hbm.at[pl.ds(base, CCH)], valv)
            pltpu.async_copy(x_hbm.at[colv], rows_v, sem).wait()

            def sbody(e16, cc):
                vs = valv[pl.ds(e16 * 16, 16)]
                for k in range(16):
                    v = vs[k]
                    e2 = e16 * 16 + k
                    for j in range(8):
                        sl = pl.ds(j * 16, 16)
                        rows_v[e2, sl] = rows_v[e2, sl] * v
                return cc

            lax.fori_loop(0, CCH // 16, sbody, 0)
            pltpu.sync_copy(rows_v, acc.at[rowv], add=True)
            return carry

        lax.fori_loop(0, NCHUNK, chunk, 0)
        plsc.subcore_barrier()
        pltpu.sync_copy(acc.at[pl.ds(s * RPT, RPT)],
                        out_hbm.at[pl.ds(s * RPT, RPT)])

    @pl.when(c == 0)
    def _():
        half(xlo_hbm, out_lo)

    @pl.when(c == 1)
    def _():
        half(xhi_hbm, out_hi)


@jax.jit
def _spmm_call(xlo, xhi, row, col, val):
    mesh = plsc.VectorSubcoreMesh(core_axis_name="c", subcore_axis_name="s")
    f = pl.kernel(
        _spmm_body,
        out_type=[jax.ShapeDtypeStruct((NP_, 128), jnp.float32),
                  jax.ShapeDtypeStruct((NP_, 128), jnp.float32)],
        mesh=mesh,
        scratch_types=[
            pltpu.VMEM_SHARED((NP_, 128), jnp.float32),
            pltpu.VMEM((ZR, 128), jnp.float32),
            pltpu.VMEM((CCH,), jnp.int32),
            pltpu.VMEM((CCH,), jnp.int32),
            pltpu.VMEM((CCH,), jnp.float32),
            pltpu.VMEM((CCH, 128), jnp.float32),
            pltpu.SemaphoreType.DMA,
        ],
    )
    return f(xlo, xhi, row, col, val)


def _mm_kernel(act, aux, blk, x_ref, w_ref, b_ref, a_ref, y_ref, ya_ref=None):
    y = jnp.dot(x_ref[...], w_ref[...], preferred_element_type=jnp.float32)
    y = y + b_ref[...]
    if act == "relu":
        y = jnp.maximum(y, 0.0)
    y_ref[...] = y
    if aux:
        ya_ref[...] = jnp.sum(y * a_ref[...], axis=1).reshape(1, 1, blk)


def _mm(x, w, b, act=None, aux_a=None):
    r = x.shape[0]
    blk = min(BLK, r)
    grid = r // blk
    aux = aux_a is not None
    a2 = aux_a.reshape(1, D) if aux else jnp.zeros((1, D), jnp.float32)
    out_shape = [jax.ShapeDtypeStruct((r, D), jnp.float32)]
    out_specs = [pl.BlockSpec((blk, D), lambda i: (i, 0))]
    if aux:
        out_shape.append(jax.ShapeDtypeStruct((grid, 1, blk), jnp.float32))
        out_specs.append(pl.BlockSpec((1, 1, blk), lambda i: (i, 0, 0)))
    res = pl.pallas_call(
        functools.partial(_mm_kernel, act, aux, blk),
        grid=(grid,),
        in_specs=[
            pl.BlockSpec((blk, D), lambda i: (i, 0)),
            pl.BlockSpec((D, D), lambda i: (0, 0)),
            pl.BlockSpec((1, D), lambda i: (0, 0)),
            pl.BlockSpec((1, D), lambda i: (0, 0)),
        ],
        out_specs=out_specs,
        out_shape=out_shape,
    )(x, w, b.reshape(1, D), a2)
    return res if aux else res[0]


def _onehot(seg, width):
    cols = lax.broadcasted_iota(jnp.int32, (BLK, width), 1)
    return seg[:, None] == cols


def _seg_kernel(width, grid, act, x_ref, seg_ref, o_ref):
    i = pl.program_id(0)
    seg = seg_ref[0, 0, :]
    rows = lax.broadcasted_iota(jnp.int32, (width, BLK), 0)
    ohT = jnp.where(rows == seg[None, :], 1.0, 0.0)
    y = jnp.dot(ohT, x_ref[...], preferred_element_type=jnp.float32)

    @pl.when(i == 0)
    def _():
        o_ref[...] = y

    @pl.when(i > 0)
    def _():
        o_ref[...] = o_ref[...] + y

    if act == "relu":
        @pl.when(i == grid - 1)
        def _():
            o_ref[...] = jnp.maximum(o_ref[...], 0.0)


def _seg_sum(x, seg3d, width, act=None):
    grid = x.shape[0] // BLK
    return pl.pallas_call(
        functools.partial(_seg_kernel, width, grid, act),
        grid=(grid,),
        in_specs=[
            pl.BlockSpec((BLK, D), lambda i: (i, 0)),
            pl.BlockSpec((1, 1, BLK), lambda i: (i, 0, 0)),
        ],
        out_specs=pl.BlockSpec((width, D), lambda i: (0, 0)),
        out_shape=jax.ShapeDtypeStruct((width, D), jnp.float32),
    )(x, seg3d)


def _e_kernel(width, es_ref, ed_ref, seg_ref, e_ref, emax_ref):
    i = pl.program_id(0)
    seg = seg_ref[0, 0, :]
    oh = _onehot(seg, width)
    edg = jnp.sum(jnp.where(oh, ed_ref[...], 0.0), axis=1)
    e = es_ref[0, 0, :] + edg
    e = jnp.where(e >= 0.0, e, 0.01 * e)
    e_ref[...] = e.reshape(1, 1, BLK)
    blkmax = jnp.max(jnp.where(oh, e[:, None], -1e30), axis=0).reshape(1, width)

    @pl.when(i == 0)
    def _():
        emax_ref[...] = blkmax

    @pl.when(i > 0)
    def _():
        emax_ref[...] = jnp.maximum(emax_ref[...], blkmax)


def _e_stage(es3d, ed_row, seg3d, width):
    grid = es3d.shape[0]
    return pl.pallas_call(
        functools.partial(_e_kernel, width),
        grid=(grid,),
        in_specs=[
            pl.BlockSpec((1, 1, BLK), lambda i: (i, 0, 0)),
            pl.BlockSpec((1, width), lambda i: (0, 0)),
            pl.BlockSpec((1, 1, BLK), lambda i: (i, 0, 0)),
        ],
        out_specs=[
            pl.BlockSpec((1, 1, BLK), lambda i: (i, 0, 0)),
            pl.BlockSpec((1, width), lambda i: (0, 0)),
        ],
        out_shape=[
            jax.ShapeDtypeStruct((grid, 1, BLK), jnp.float32),
            jax.ShapeDtypeStruct((1, width), jnp.float32),
        ],
    )(es3d, ed_row, seg3d)


def _a_kernel(width, e_ref, emax_ref, seg_ref, a_ref, den_ref):
    i = pl.program_id(0)
    seg = seg_ref[0, 0, :]
    oh = _onehot(seg, width)
    emaxg = jnp.sum(jnp.where(oh, emax_ref[...], 0.0), axis=1)
    a = jnp.exp(e_ref[0, 0, :] - emaxg)
    a_ref[...] = a.reshape(1, 1, BLK)
    blksum = jnp.sum(jnp.where(oh, a[:, None], 0.0), axis=0).reshape(1, width)

    @pl.when(i == 0)
    def _():
        den_ref[...] = blksum

    @pl.when(i > 0)
    def _():
        den_ref[...] = den_ref[...] + blksum


def _a_stage(e3d, emax, seg3d, width):
    grid = e3d.shape[0]
    return pl.pallas_call(
        functools.partial(_a_kernel, width),
        grid=(grid,),
        in_specs=[
            pl.BlockSpec((1, 1, BLK), lambda i: (i, 0, 0)),
            pl.BlockSpec((1, width), lambda i: (0, 0)),
            pl.BlockSpec((1, 1, BLK), lambda i: (i, 0, 0)),
        ],
        out_specs=[
            pl.BlockSpec((1, 1, BLK), lambda i: (i, 0, 0)),
            pl.BlockSpec((1, width), lambda i: (0, 0)),
        ],
        out_shape=[
            jax.ShapeDtypeStruct((grid, 1, BLK), jnp.float32),
            jax.ShapeDtypeStruct((1, width), jnp.float32),
        ],
    )(e3d, emax, seg3d)


def _pool_kernel(width, grid, a_ref, den_ref, hs_ref, seg_ref, b_ref, o_ref):
    i = pl.program_id(0)
    seg = seg_ref[0, 0, :]
    oh = _onehot(seg, width)
    deng = jnp.sum(jnp.where(oh, den_ref[...], 0.0), axis=1)
    alpha = a_ref[0, 0, :] / (deng + 1e-16)
    rows = lax.broadcasted_iota(jnp.int32, (width, BLK), 0)
    ohT = jnp.where(rows == seg[None, :], 1.0, 0.0)
    y = jnp.dot(ohT, alpha[:, None] * hs_ref[...],
                preferred_element_type=jnp.float32)

    @pl.when(i == 0)
    def _():
        o_ref[...] = y

    @pl.when(i > 0)
    def _():
        o_ref[...] = o_ref[...] + y

    @pl.when(i == grid - 1)
    def _():
        z = o_ref[...] + b_ref[...]
        z = jnp.where(z > 0.0, z, jnp.exp(jnp.minimum(z, 0.0)) - 1.0)
        o_ref[...] = jnp.maximum(z, 0.0)


def _pool_stage(a3d, denom, hs, seg3d, b, width):
    grid = a3d.shape[0]
    return pl.pallas_call(
        functools.partial(_pool_kernel, width, grid),
        grid=(grid,),
        in_specs=[
            pl.BlockSpec((1, 1, BLK), lambda i: (i, 0, 0)),
            pl.BlockSpec((1, width), lambda i: (0, 0)),
            pl.BlockSpec((BLK, D), lambda i: (i, 0)),
            pl.BlockSpec((1, 1, BLK), lambda i: (i, 0, 0)),
            pl.BlockSpec((1, D), lambda i: (0, 0)),
        ],
        out_specs=pl.BlockSpec((width, D), lambda i: (0, 0)),
        out_shape=jax.ShapeDtypeStruct((width, D), jnp.float32),
    )(a3d, denom, hs, seg3d, b.reshape(1, D))


def _mlp_kernel(x_ref, p1_ref, pb1_ref, p2_ref, pb2_ref, o_ref):
    p = jnp.maximum(x_ref[...] @ p1_ref[...] + pb1_ref[...], 0.0)
    o_ref[...] = p @ p2_ref[...] + pb2_ref[...]


def kernel(node_attr, adj_index, adj_value, tt_node_batch, tt_graph_batch,
           W1, b1, W2, b2,
           g1Ws, g1Wd, g1as, g1ad, g1b,
           g2Ws, g2Wd, g2as, g2ad, g2b,
           P1, pb1, P2, pb2):
    row = adj_index[0]
    col = adj_index[1]
    zb = jnp.zeros((D,), jnp.float32)

    ttb = jnp.pad(tt_node_batch, (0, NP_ - N), constant_values=3000)
    ttb3d = ttb.reshape(NP_ // BLK, 1, BLK)
    tgb = jnp.pad(tt_graph_batch, (0, TP_ - T), constant_values=3000)
    tgb3d = tgb.reshape(TP_ // BLK, 1, BLK)

    s1lo, s1hi = _spmm_call(node_attr[:, :128], node_attr[:, 128:],
                            row, col, adj_value)
    s1p = jnp.concatenate([s1lo, s1hi], axis=1)
    hp = _mm(s1p, W1, b1, act="relu")

    h = hp[:N]
    s2lo, s2hi = _spmm_call(h[:, :128], h[:, 128:], row, col, adj_value)
    s2p = jnp.concatenate([s2lo, s2hi], axis=1)
    h1p = _mm(s2p, W2, b2, act="relu")

    tt_attr = _seg_sum(h1p, ttb3d, TP_, act="relu")
    hs, es3d = _mm(h1p, g1Ws, zb, aux_a=g1as)
    _, ed3d = _mm(tt_attr, g1Wd, zb, aux_a=g1ad)
    ed_row = ed3d.reshape(1, TP_)
    e3d, emax = _e_stage(es3d, ed_row, ttb3d, TP_)
    a3d, denom = _a_stage(e3d, emax, ttb3d, TP_)
    tt_embd = _pool_stage(a3d, denom, hs, ttb3d, g1b, TP_)

    mol_attr = _seg_sum(tt_embd, tgb3d, M, act="relu")
    hs2, es2_3d = _mm(tt_embd, g2Ws, zb, aux_a=g2as)
    _, ed2_3d = _mm(mol_attr, g2Wd, zb, aux_a=g2ad)
    ed2_row = ed2_3d.reshape(1, M)
    e2_3d, e2max = _e_stage(es2_3d, ed2_row, tgb3d, M)
    a2_3d, denom2 = _a_stage(e2_3d, e2max, tgb3d, M)
    mol_embd = _pool_stage(a2_3d, denom2, hs2, tgb3d, g2b, M)

    y = pl.pallas_call(
        _mlp_kernel,
        out_shape=jax.ShapeDtypeStruct((M, 1), jnp.float32),
    )(mol_embd, P1, pb1.reshape(1, 128), P2, pb2.reshape(1, 1))
    return y


# speedup vs baseline: 3.3885x; 1.2397x over previous
import functools

import jax
import jax.numpy as jnp
from jax import lax
from jax.experimental import pallas as pl
from jax.experimental.pallas import tpu as pltpu
from jax.experimental.pallas import tpu_sc as plsc

N = 10000
E = 160000
D = 256
T = 2000
M = 128

NP_ = 10240
TP_ = 2048
BLK = 256

NS = 16
EP = 163840
EPT = EP // NS
CCH = 128
NCHUNK = EPT // CCH
RPT = NP_ // NS
ZR = 32


def _spmm_body(xlo_hbm, xhi_hbm, row_hbm, col_hbm, val_hbm, out_lo, out_hi,
               acc, zbuf,
               colv0, colv1, colv2, colv3,
               rowv0, rowv1, rowv2, rowv3,
               valv0, valv1, valv2, valv3,
               rowsA, rowsB,
               isem0, isem1, isem2, isem3,
               gsemA, gsemB, ssemA, ssemB):
    c = lax.axis_index("c")
    s = lax.axis_index("s")
    colv = [colv0, colv1, colv2, colv3]
    rowv = [rowv0, rowv1, rowv2, rowv3]
    valv = [valv0, valv1, valv2, valv3]
    isem = [isem0, isem1, isem2, isem3]
    rows = [rowsA, rowsB]
    gsem = [gsemA, gsemB]
    ssem = [ssemA, ssemB]

    def zrow(i, carry):
        for j in range(8):
            zbuf[i, pl.ds(j * 16, 16)] = jnp.zeros((16,), jnp.float32)
        return carry

    lax.fori_loop(0, ZR, zrow, 0)
    for k in range(RPT // ZR):
        pltpu.sync_copy(zbuf, acc.at[pl.ds(s * RPT + k * ZR, ZR)])
    plsc.subcore_barrier()

    def half(x_hbm, out_hbm):
        def issue_idx(chunk, slot):
            base = s * EPT + chunk * CCH
            pltpu.async_copy(col_hbm.at[pl.ds(base, CCH)], colv[slot], isem[slot])
            pltpu.async_copy(row_hbm.at[pl.ds(base, CCH)], rowv[slot], isem[slot])
            pltpu.async_copy(val_hbm.at[pl.ds(base, CCH)], valv[slot], isem[slot])

        def wait_idx(slot):
            pltpu.make_async_copy(col_hbm.at[pl.ds(0, CCH)], colv[slot], isem[slot]).wait()
            pltpu.make_async_copy(row_hbm.at[pl.ds(0, CCH)], rowv[slot], isem[slot]).wait()
            pltpu.make_async_copy(val_hbm.at[pl.ds(0, CCH)], valv[slot], isem[slot]).wait()

        def issue_gather(slot, rp):
            pltpu.async_copy(x_hbm.at[colv[slot]], rows[rp], gsem[rp])

        def wait_gather(rp):
            pltpu.make_async_copy(x_hbm.at[colv[0]], rows[rp], gsem[rp]).wait()

        def issue_scatter(slot, rp):
            pltpu.async_copy(rows[rp], acc.at[rowv[slot]], ssem[rp], add=True)

        def wait_scatter(rp):
            pltpu.make_async_copy(rows[rp], acc.at[rowv[0]], ssem[rp]).wait()

        def scale(slot, rp):
            def sbody(g, cc):
                vs = valv[slot][pl.ds(g * 16, 16)]
                for k in range(16):
                    v = vs[k]
                    e2 = g * 16 + k
                    for j in range(8):
                        sl = pl.ds(j * 16, 16)
                        rows[rp][e2, sl] = rows[rp][e2, sl] * v
                return cc

            lax.fori_loop(0, CCH // 16, sbody, 0)

        for j in range(3):
            issue_idx(j, j)
        wait_idx(0)
        issue_gather(0, 0)

        def outer(i2, cc):
            for b4 in range(4):
                i = i2 * 4 + b4
                rp = b4 % 2
                nrp = 1 - rp
                nslot = (b4 + 1) % 4
                pslot = (b4 + 3) % 4

                @pl.when(i >= 1)
                def _():
                    wait_scatter(nrp)

                @pl.when(i + 3 < NCHUNK)
                def _():
                    issue_idx(i + 3, pslot)

                @pl.when(i + 1 < NCHUNK)
                def _():
                    wait_idx(nslot)
                    issue_gather(nslot, nrp)

                wait_gather(rp)
                scale(b4, rp)
                issue_scatter(b4, rp)
            return cc

        lax.fori_loop(0, NCHUNK // 4, outer, 0)
        wait_scatter(1)
        plsc.subcore_barrier()
        pltpu.sync_copy(acc.at[pl.ds(s * RPT, RPT)],
                        out_hbm.at[pl.ds(s * RPT, RPT)])

    @pl.when(c == 0)
    def _():
        half(xlo_hbm, out_lo)

    @pl.when(c == 1)
    def _():
        half(xhi_hbm, out_hi)


@jax.jit
def _spmm_call(xlo, xhi, row, col, val):
    mesh = plsc.VectorSubcoreMesh(core_axis_name="c", subcore_axis_name="s")
    idx_scr = ([pltpu.VMEM((CCH,), jnp.int32) for _ in range(8)]
               + [pltpu.VMEM((CCH,), jnp.float32) for _ in range(4)])
    f = pl.kernel(
        _spmm_body,
        out_type=[jax.ShapeDtypeStruct((NP_, 128), jnp.float32),
                  jax.ShapeDtypeStruct((NP_, 128), jnp.float32)],
        mesh=mesh,
        scratch_types=[
            pltpu.VMEM_SHARED((NP_, 128), jnp.float32),
            pltpu.VMEM((ZR, 128), jnp.float32),
        ] + idx_scr + [
            pltpu.VMEM((CCH, 128), jnp.float32),
            pltpu.VMEM((CCH, 128), jnp.float32),
        ] + [pltpu.SemaphoreType.DMA] * 8,
    )
    return f(xlo, xhi, row, col, val)


def _make_seg_sc(R, W, scaled):
    RPT2 = R // NS
    SCH = min(128, RPT2)
    NCH = RPT2 // SCH
    WPT = W // NS

    def body(*refs):
        if scaled:
            (xlo_hbm, xhi_hbm, seg_hbm, scl_hbm, out_lo, out_hi,
             acc, zbuf, segv, sclv, rows_v) = refs
        else:
            (xlo_hbm, xhi_hbm, seg_hbm, out_lo, out_hi,
             acc, zbuf, segv, rows_v) = refs
            sclv = None
        c = lax.axis_index("c")
        s = lax.axis_index("s")

        def zrow(i, carry):
            for j in range(8):
                zbuf[i, pl.ds(j * 16, 16)] = jnp.zeros((16,), jnp.float32)
            return carry

        zr = min(ZR, WPT)
        lax.fori_loop(0, zr, zrow, 0)
        for k in range(WPT // zr):
            pltpu.sync_copy(zbuf.at[pl.ds(0, zr)],
                            acc.at[pl.ds(s * WPT + k * zr, zr)])
        plsc.subcore_barrier()

        def half(x_hbm, out_hbm):
            def chunk(i, carry):
                base = s * RPT2 + i * SCH
                pltpu.sync_copy(seg_hbm.at[pl.ds(base, SCH)], segv)
                pltpu.sync_copy(x_hbm.at[pl.ds(base, SCH)], rows_v)
                if scaled:
                    pltpu.sync_copy(scl_hbm.at[pl.ds(base, SCH)], sclv)

                    def sbody(g, cc):
                        vs = sclv[pl.ds(g * 16, 16)]
                        for k in range(16):
                            v = vs[k]
                            e2 = g * 16 + k
                            for j in range(8):
                                sl = pl.ds(j * 16, 16)
                                rows_v[e2, sl] = rows_v[e2, sl] * v
                        return cc

                    lax.fori_loop(0, SCH // 16, sbody, 0)
                pltpu.sync_copy(rows_v, acc.at[segv], add=True)
                return carry

            lax.fori_loop(0, NCH, chunk, 0)
            plsc.subcore_barrier()
            pltpu.sync_copy(acc.at[pl.ds(s * WPT, WPT)],
                            out_hbm.at[pl.ds(s * WPT, WPT)])

        @pl.when(c == 0)
        def _():
            half(xlo_hbm, out_lo)

        @pl.when(c == 1)
        def _():
            half(xhi_hbm, out_hi)

    mesh = plsc.VectorSubcoreMesh(core_axis_name="c", subcore_axis_name="s")
    scr = [
        pltpu.VMEM_SHARED((W, 128), jnp.float32),
        pltpu.VMEM((min(ZR, WPT), 128), jnp.float32),
        pltpu.VMEM((SCH,), jnp.int32),
    ]
    if scaled:
        scr.insert(3, pltpu.VMEM((SCH,), jnp.float32))
    scr.append(pltpu.VMEM((SCH, 128), jnp.float32))
    f = pl.kernel(
        body,
        out_type=[jax.ShapeDtypeStruct((W, 128), jnp.float32),
                  jax.ShapeDtypeStruct((W, 128), jnp.float32)],
        mesh=mesh,
        scratch_types=scr,
    )
    return jax.jit(f)


@jax.jit
def _seg_sc_tt(xlo, xhi, seg):
    return _make_seg_sc(NP_, TP_, False)(xlo, xhi, seg)


@jax.jit
def _seg_sc_tt_scaled(xlo, xhi, seg, scl):
    return _make_seg_sc(NP_, TP_, True)(xlo, xhi, seg, scl)


@jax.jit
def _seg_sc_mol(xlo, xhi, seg):
    return _make_seg_sc(TP_, M, False)(xlo, xhi, seg)


@jax.jit
def _seg_sc_mol_scaled(xlo, xhi, seg, scl):
    return _make_seg_sc(TP_, M, True)(xlo, xhi, seg, scl)


def _mm_kernel(act, aux, blk, act_in, x_ref, w_ref, b_ref, a_ref, y_ref,
               ya_ref=None):
    x = x_ref[...]
    if act_in == "relu":
        x = jnp.maximum(x, 0.0)
    y = jnp.dot(x, w_ref[...], preferred_element_type=jnp.float32)
    y = y + b_ref[...]
    if act == "relu":
        y = jnp.maximum(y, 0.0)
    y_ref[...] = y
    if aux:
        ya = jnp.dot(y, a_ref[...], preferred_element_type=jnp.float32)
        ya_ref[...] = ya.reshape(1, 1, blk)


def _mm(x, w, b, act=None, aux_a=None, act_in=None):
    r = x.shape[0]
    blk = min(BLK, r)
    grid = r // blk
    aux = aux_a is not None
    a2 = aux_a.reshape(D, 1) if aux else jnp.zeros((D, 1), jnp.float32)
    out_shape = [jax.ShapeDtypeStruct((r, D), jnp.float32)]
    out_specs = [pl.BlockSpec((blk, D), lambda i: (i, 0))]
    if aux:
        out_shape.append(jax.ShapeDtypeStruct((grid, 1, blk), jnp.float32))
        out_specs.append(pl.BlockSpec((1, 1, blk), lambda i: (i, 0, 0)))
    res = pl.pallas_call(
        functools.partial(_mm_kernel, act, aux, blk, act_in),
        grid=(grid,),
        in_specs=[
            pl.BlockSpec((blk, D), lambda i: (i, 0)),
            pl.BlockSpec((D, D), lambda i: (0, 0)),
            pl.BlockSpec((1, D), lambda i: (0, 0)),
            pl.BlockSpec((D, 1), lambda i: (0, 0)),
        ],
        out_specs=out_specs,
        out_shape=out_shape,
    )(x, w, b.reshape(1, D), a2)
    return res if aux else res[0]


def _onehot(seg, width):
    cols = lax.broadcasted_iota(jnp.int32, (BLK, width), 1)
    return seg[:, None] == cols


def _seg_kernel(width, grid, act, x_ref, seg_ref, o_ref):
    i = pl.program_id(0)
    seg = seg_ref[0, 0, :]
    rows = lax.broadcasted_iota(jnp.int32, (width, BLK), 0)
    ohT = jnp.where(rows == seg[None, :], 1.0, 0.0)
    y = jnp.dot(ohT, x_ref[...], preferred_element_type=jnp.float32,
                precision=lax.Precision.HIGHEST)

    @pl.when(i == 0)
    def _():
        o_ref[...] = y

    @pl.when(i > 0)
    def _():
        o_ref[...] = o_ref[...] + y

    if act == "relu":
        @pl.when(i == grid - 1)
        def _():
            o_ref[...] = jnp.maximum(o_ref[...], 0.0)


def _seg_sum(x, seg3d, width, act=None):
    grid = x.shape[0] // BLK
    return pl.pallas_call(
        functools.partial(_seg_kernel, width, grid, act),
        grid=(grid,),
        in_specs=[
            pl.BlockSpec((BLK, D), lambda i: (i, 0)),
            pl.BlockSpec((1, 1, BLK), lambda i: (i, 0, 0)),
        ],
        out_specs=pl.BlockSpec((width, D), lambda i: (0, 0)),
        out_shape=jax.ShapeDtypeStruct((width, D), jnp.float32),
    )(x, seg3d)


def _e_kernel(width, es_ref, ed_ref, seg_ref, e_ref, emax_ref):
    i = pl.program_id(0)
    seg = seg_ref[0, 0, :]
    oh = _onehot(seg, width)
    edg = jnp.sum(jnp.where(oh, ed_ref[...], 0.0), axis=1)
    e = es_ref[0, 0, :] + edg
    e = jnp.where(e >= 0.0, e, 0.01 * e)
    e_ref[...] = e.reshape(1, 1, BLK)
    blkmax = jnp.max(jnp.where(oh, e[:, None], -1e30), axis=0).reshape(1, width)

    @pl.when(i == 0)
    def _():
        emax_ref[...] = blkmax

    @pl.when(i > 0)
    def _():
        emax_ref[...] = jnp.maximum(emax_ref[...], blkmax)


def _e_stage(es3d, ed_row, seg3d, width):
    grid = es3d.shape[0]
    return pl.pallas_call(
        functools.partial(_e_kernel, width),
        grid=(grid,),
        in_specs=[
            pl.BlockSpec((1, 1, BLK), lambda i: (i, 0, 0)),
            pl.BlockSpec((1, width), lambda i: (0, 0)),
            pl.BlockSpec((1, 1, BLK), lambda i: (i, 0, 0)),
        ],
        out_specs=[
            pl.BlockSpec((1, 1, BLK), lambda i: (i, 0, 0)),
            pl.BlockSpec((1, width), lambda i: (0, 0)),
        ],
        out_shape=[
            jax.ShapeDtypeStruct((grid, 1, BLK), jnp.float32),
            jax.ShapeDtypeStruct((1, width), jnp.float32),
        ],
    )(es3d, ed_row, seg3d)


def _a_kernel(width, e_ref, emax_ref, seg_ref, a_ref, den_ref):
    i = pl.program_id(0)
    seg = seg_ref[0, 0, :]
    oh = _onehot(seg, width)
    emaxg = jnp.sum(jnp.where(oh, emax_ref[...], 0.0), axis=1)
    a = jnp.exp(e_ref[0, 0, :] - emaxg)
    a_ref[...] = a.reshape(1, 1, BLK)
    blksum = jnp.sum(jnp.where(oh, a[:, None], 0.0), axis=0).reshape(1, width)

    @pl.when(i == 0)
    def _():
        den_ref[...] = blksum

    @pl.when(i > 0)
    def _():
        den_ref[...] = den_ref[...] + blksum


def _a_stage(e3d, emax, seg3d, width):
    grid = e3d.shape[0]
    return pl.pallas_call(
        functools.partial(_a_kernel, width),
        grid=(grid,),
        in_specs=[
            pl.BlockSpec((1, 1, BLK), lambda i: (i, 0, 0)),
            pl.BlockSpec((1, width), lambda i: (0, 0)),
            pl.BlockSpec((1, 1, BLK), lambda i: (i, 0, 0)),
        ],
        out_specs=[
            pl.BlockSpec((1, 1, BLK), lambda i: (i, 0, 0)),
            pl.BlockSpec((1, width), lambda i: (0, 0)),
        ],
        out_shape=[
            jax.ShapeDtypeStruct((grid, 1, BLK), jnp.float32),
            jax.ShapeDtypeStruct((1, width), jnp.float32),
        ],
    )(e3d, emax, seg3d)


def _alpha_kernel(width, a_ref, den_ref, seg_ref, al_ref):
    seg = seg_ref[0, 0, :]
    oh = _onehot(seg, width)
    deng = jnp.sum(jnp.where(oh, den_ref[...], 0.0), axis=1)
    al_ref[...] = (a_ref[0, 0, :] / (deng + 1e-16)).reshape(1, 1, BLK)


def _alpha_stage(a3d, denom, seg3d, width):
    grid = a3d.shape[0]
    return pl.pallas_call(
        functools.partial(_alpha_kernel, width),
        grid=(grid,),
        in_specs=[
            pl.BlockSpec((1, 1, BLK), lambda i: (i, 0, 0)),
            pl.BlockSpec((1, width), lambda i: (0, 0)),
            pl.BlockSpec((1, 1, BLK), lambda i: (i, 0, 0)),
        ],
        out_specs=pl.BlockSpec((1, 1, BLK), lambda i: (i, 0, 0)),
        out_shape=jax.ShapeDtypeStruct((grid, 1, BLK), jnp.float32),
    )(a3d, denom, seg3d)


def _ewise_kernel(x_ref, b_ref, o_ref):
    z = x_ref[...] + b_ref[...]
    z = jnp.where(z > 0.0, z, jnp.exp(jnp.minimum(z, 0.0)) - 1.0)
    o_ref[...] = jnp.maximum(z, 0.0)


def _ewise(x, b):
    r = x.shape[0]
    blk = min(BLK, r)
    return pl.pallas_call(
        _ewise_kernel,
        grid=(r // blk,),
        in_specs=[
            pl.BlockSpec((blk, D), lambda i: (i, 0)),
            pl.BlockSpec((1, D), lambda i: (0, 0)),
        ],
        out_specs=pl.BlockSpec((blk, D), lambda i: (i, 0)),
        out_shape=jax.ShapeDtypeStruct((r, D), jnp.float32),
    )(x, b.reshape(1, D))


def _pool_kernel(width, grid, a_ref, den_ref, hs_ref, seg_ref, b_ref, o_ref):
    i = pl.program_id(0)
    seg = seg_ref[0, 0, :]
    oh = _onehot(seg, width)
    deng = jnp.sum(jnp.where(oh, den_ref[...], 0.0), axis=1)
    alpha = a_ref[0, 0, :] / (deng + 1e-16)
    rows = lax.broadcasted_iota(jnp.int32, (width, BLK), 0)
    ohT = jnp.where(rows == seg[None, :], 1.0, 0.0)
    y = jnp.dot(ohT, alpha[:, None] * hs_ref[...],
                preferred_element_type=jnp.float32,
                precision=lax.Precision.HIGHEST)

    @pl.when(i == 0)
    def _():
        o_ref[...] = y

    @pl.when(i > 0)
    def _():
        o_ref[...] = o_ref[...] + y

    @pl.when(i == grid - 1)
    def _():
        z = o_ref[...] + b_ref[...]
        z = jnp.where(z > 0.0, z, jnp.exp(jnp.minimum(z, 0.0)) - 1.0)
        o_ref[...] = jnp.maximum(z, 0.0)


def _pool_stage(a3d, denom, hs, seg3d, b, width):
    grid = a3d.shape[0]
    return pl.pallas_call(
        functools.partial(_pool_kernel, width, grid),
        grid=(grid,),
        in_specs=[
            pl.BlockSpec((1, 1, BLK), lambda i: (i, 0, 0)),
            pl.BlockSpec((1, width), lambda i: (0, 0)),
            pl.BlockSpec((BLK, D), lambda i: (i, 0)),
            pl.BlockSpec((1, 1, BLK), lambda i: (i, 0, 0)),
            pl.BlockSpec((1, D), lambda i: (0, 0)),
        ],
        out_specs=pl.BlockSpec((width, D), lambda i: (0, 0)),
        out_shape=jax.ShapeDtypeStruct((width, D), jnp.float32),
    )(a3d, denom, hs, seg3d, b.reshape(1, D))


def _mlp_kernel(x_ref, p1_ref, pb1_ref, p2_ref, pb2_ref, o_ref):
    p = jnp.maximum(
        jnp.dot(x_ref[...], p1_ref[...],
                preferred_element_type=jnp.float32) + pb1_ref[...], 0.0)
    o_ref[...] = jnp.dot(p, p2_ref[...],
                         preferred_element_type=jnp.float32) + pb2_ref[...]


def kernel(node_attr, adj_index, adj_value, tt_node_batch, tt_graph_batch,
           W1, b1, W2, b2,
           g1Ws, g1Wd, g1as, g1ad, g1b,
           g2Ws, g2Wd, g2as, g2ad, g2b,
           P1, pb1, P2, pb2):
    row = jnp.pad(adj_index[0], (0, EP - E))
    col = jnp.pad(adj_index[1], (0, EP - E))
    adj_value = jnp.pad(adj_value, (0, EP - E))
    zb = jnp.zeros((D,), jnp.float32)

    ttb = jnp.pad(tt_node_batch, (0, NP_ - N), constant_values=3000)
    ttb3d = ttb.reshape(NP_ // BLK, 1, BLK)
    tgb = jnp.pad(tt_graph_batch, (0, TP_ - T), constant_values=3000)
    tgb3d = tgb.reshape(TP_ // BLK, 1, BLK)
    ttb_sc = jnp.pad(tt_node_batch, (0, NP_ - N), constant_values=TP_ - 1)
    tgb_sc = jnp.pad(tt_graph_batch, (0, TP_ - T), constant_values=M - 1)

    s1lo, s1hi = _spmm_call(node_attr[:, :128], node_attr[:, 128:],
                            row, col, adj_value)
    s1p = jnp.concatenate([s1lo, s1hi], axis=1)
    hp = _mm(s1p, W1, b1, act="relu")

    h = hp[:N]
    s2lo, s2hi = _spmm_call(h[:, :128], h[:, 128:], row, col, adj_value)
    s2p = jnp.concatenate([s2lo, s2hi], axis=1)
    h1p = _mm(s2p, W2, b2, act="relu")

    tslo, tshi = _seg_sc_tt(h1p[:, :128], h1p[:, 128:], ttb_sc)
    tt_sum = jnp.concatenate([tslo, tshi], axis=1)
    hs, es3d = _mm(h1p, g1Ws, zb, aux_a=g1as)
    _, ed3d = _mm(tt_sum, g1Wd, zb, aux_a=g1ad, act_in="relu")
    ed_row = ed3d.reshape(1, TP_)
    e3d, emax = _e_stage(es3d, ed_row, ttb3d, TP_)
    a3d, denom = _a_stage(e3d, emax, ttb3d, TP_)
    alpha = _alpha_stage(a3d, denom, ttb3d, TP_).reshape(NP_)
    p1lo, p1hi = _seg_sc_tt_scaled(hs[:, :128], hs[:, 128:], ttb_sc, alpha)
    tt_embd = _ewise(jnp.concatenate([p1lo, p1hi], axis=1), g1b)

    mslo, mshi = _seg_sc_mol(tt_embd[:, :128], tt_embd[:, 128:], tgb_sc)
    mol_sum = jnp.concatenate([mslo, mshi], axis=1)
    hs2, es2_3d = _mm(tt_embd, g2Ws, zb, aux_a=g2as)
    _, ed2_3d = _mm(mol_sum, g2Wd, zb, aux_a=g2ad, act_in="relu")
    ed2_row = ed2_3d.reshape(1, M)
    e2_3d, e2max = _e_stage(es2_3d, ed2_row, tgb3d, M)
    a2_3d, denom2 = _a_stage(e2_3d, e2max, tgb3d, M)
    alpha2 = _alpha_stage(a2_3d, denom2, tgb3d, M).reshape(TP_)
    p2lo, p2hi = _seg_sc_mol_scaled(hs2[:, :128], hs2[:, 128:], tgb_sc, alpha2)
    mol_embd = _ewise(jnp.concatenate([p2lo, p2hi], axis=1), g2b)

    y = pl.pallas_call(
        _mlp_kernel,
        out_shape=jax.ShapeDtypeStruct((M, 1), jnp.float32),
    )(mol_embd, P1, pb1.reshape(1, 128), P2, pb2.reshape(1, 1))
    return y
